# baseline (speedup 1.0000x reference)
import functools

import jax
import jax.numpy as jnp
from jax import lax
from jax.experimental import pallas as pl
from jax.experimental.pallas import tpu as pltpu
from jax.experimental.pallas import tpu_sc as plsc

B = 16384
D = 32
N = 1000000
HIDDEN = 64
PD = 2 * HIDDEN
CB = 12288
SUB = CB // 2
PGRID = -(-N // CB)
ROWS = PGRID * SUB
NC = 2
NS = 16
NW = NC * NS
BPW = B // NW
CHUNK = 128
NCH = BPW // CHUNK

MB = 2048


def _premul_body(t_ref, w_ref, out_ref):
    cdims = (((0,), (0,)), ((), ()))
    t = t_ref[...].astype(jnp.bfloat16)
    w = w_ref[...].astype(jnp.bfloat16)
    out_ref[:, :HIDDEN] = lax.dot_general(
        t[:, :SUB], w, cdims, preferred_element_type=jnp.float32)
    out_ref[:, HIDDEN:] = lax.dot_general(
        t[:, SUB:], w, cdims, preferred_element_type=jnp.float32)


def _premul(tabT, w):
    return pl.pallas_call(
        _premul_body,
        grid=(PGRID,),
        in_specs=[
            pl.BlockSpec((D, CB), lambda j: (0, j)),
            pl.BlockSpec((D, HIDDEN), lambda j: (0, 0)),
        ],
        out_specs=pl.BlockSpec((SUB, PD), lambda j: (j, 0)),
        out_shape=jax.ShapeDtypeStruct((ROWS, PD), jnp.float32),
        compiler_params=pltpu.CompilerParams(
            fuse_transposed_lhs_in_matmul=True),
    )(tabT, w)


def _sc_gather(gidx2d, tab):
    mesh = plsc.VectorSubcoreMesh(core_axis_name="c", subcore_axis_name="s")

    @functools.partial(
        pl.kernel,
        mesh=mesh,
        out_type=jax.ShapeDtypeStruct((B, PD), jnp.float32),
        scratch_types=[
            pltpu.VMEM((NCH, CHUNK), jnp.int32),
            pltpu.VMEM((2, CHUNK, PD), jnp.float32),
            pltpu.SemaphoreType.DMA,
            pltpu.SemaphoreType.DMA,
            pltpu.SemaphoreType.DMA,
            pltpu.SemaphoreType.DMA,
        ],
    )
    def gather_kernel(idx_hbm, tab_hbm, out_hbm,
                      idx_v, buf_v, gsem0, gsem1, osem0, osem1):
        wid = lax.axis_index("s") * NC + lax.axis_index("c")
        pltpu.sync_copy(idx_hbm.at[pl.ds(wid * NCH, NCH)], idx_v)
        base = wid * BPW
        gsems = (gsem0, gsem1)
        osems = (osem0, osem1)

        def gath(j):
            return pltpu.async_copy(
                tab_hbm.at[idx_v.at[j]], buf_v.at[j % 2], gsems[j % 2])

        def out(j):
            return pltpu.async_copy(
                buf_v.at[j % 2],
                out_hbm.at[pl.ds(base + j * CHUNK, CHUNK)], osems[j % 2])

        gc = [None] * NCH
        oc = [None] * NCH
        gc[0] = gath(0)
        gc[1] = gath(1)
        for j in range(NCH):
            gc[j].wait()
            oc[j] = out(j)
            if j + 2 < NCH:
                oc[j].wait()
                gc[j + 2] = gath(j + 2)
        for j in range(NCH - 2, NCH):
            oc[j].wait()

    return gather_kernel(gidx2d, tab)


def _mlp_body(u128_ref, d128_ref, ku_ref, kd_ref, img_ref,
              w1i_ref, b1_ref, w2t_ref, b2_ref, out_ref):
    ku = ku_ref[...]
    kd = kd_ref[...]
    acc = jnp.dot(img_ref[...], w1i_ref[...], preferred_element_type=jnp.float32)
    for k in range(2):
        acc = acc + jnp.where(ku == k, u128_ref[:, k * HIDDEN:(k + 1) * HIDDEN], 0.0)
        acc = acc + jnp.where(kd == k, d128_ref[:, k * HIDDEN:(k + 1) * HIDDEN], 0.0)
    h = jnp.maximum(acc + b1_ref[...], 0.0)
    out_ref[...] = jnp.sum(h * w2t_ref[...], axis=1) + b2_ref[0]


def kernel(user_idx, deal_idx, image_vec, user_table, deal_table, W1, b1, W2, b2):
    uidx = user_idx.astype(jnp.int32)
    didx = deal_idx.astype(jnp.int32)
    ugidx2d = ((uidx // CB) * SUB + uidx % SUB).reshape(B // CHUNK, CHUNK)
    dgidx2d = ((didx // CB) * SUB + didx % SUB).reshape(B // CHUNK, CHUNK)

    w1u, w1d, w1i = W1[:D], W1[D:2 * D], W1[2 * D:]
    utw = _premul(user_table.T, w1u)
    u128 = _sc_gather(ugidx2d, utw)
    dtw = _premul(deal_table.T, w1d)
    d128 = _sc_gather(dgidx2d, dtw)

    ku2d = ((uidx // SUB) & 1).reshape(B, 1)
    kd2d = ((didx // SUB) & 1).reshape(B, 1)
    b1r = b1.reshape(1, HIDDEN)
    w2t = W2.reshape(1, HIDDEN)

    score = pl.pallas_call(
        _mlp_body,
        grid=(B // MB,),
        in_specs=[
            pl.BlockSpec((MB, PD), lambda i: (i, 0)),
            pl.BlockSpec((MB, PD), lambda i: (i, 0)),
            pl.BlockSpec((MB, 1), lambda i: (i, 0)),
            pl.BlockSpec((MB, 1), lambda i: (i, 0)),
            pl.BlockSpec((MB, D), lambda i: (i, 0)),
            pl.BlockSpec((D, HIDDEN), lambda i: (0, 0)),
            pl.BlockSpec((1, HIDDEN), lambda i: (0, 0)),
            pl.BlockSpec((1, HIDDEN), lambda i: (0, 0)),
            pl.BlockSpec(memory_space=pltpu.SMEM),
        ],
        out_specs=pl.BlockSpec((MB,), lambda i: (i,)),
        out_shape=jax.ShapeDtypeStruct((B,), jnp.float32),
    )(u128, d128, ku2d, kd2d, image_vec, w1i, b1r, w2t, b2)
    return score


# speedup vs baseline: 1.1500x; 1.1500x over previous
import functools

import jax
import jax.numpy as jnp
from jax import lax
from jax.experimental import pallas as pl
from jax.experimental.pallas import tpu as pltpu
from jax.experimental.pallas import tpu_sc as plsc

B = 16384
D = 32
N = 1000000
HIDDEN = 64
PD = 2 * HIDDEN
CB = 12288
SUB = CB // 4
PGRID = -(-N // CB)
ROWS = PGRID * SUB
NC = 2
NS = 16
NW = NC * NS
BPW = B // NW
CHUNK = 128
NCH = BPW // CHUNK

MB = 2048


def _premul_body(t_ref, w_ref, out_ref):
    cdims = (((0,), (0,)), ((), ()))
    t = t_ref[...].astype(jnp.bfloat16)
    w = w_ref[...].astype(jnp.bfloat16)
    q = []
    for k in range(4):
        r = lax.dot_general(t[:, k * SUB:(k + 1) * SUB], w, cdims,
                            preferred_element_type=jnp.float32)
        u16 = lax.bitcast_convert_type(r.astype(jnp.bfloat16), jnp.uint16)
        q.append(u16.astype(jnp.uint32))
    out_ref[:, :HIDDEN] = lax.bitcast_convert_type(
        q[0] | (q[1] << 16), jnp.int32)
    out_ref[:, HIDDEN:] = lax.bitcast_convert_type(
        q[2] | (q[3] << 16), jnp.int32)


def _premul(tabT, w):
    return pl.pallas_call(
        _premul_body,
        grid=(PGRID,),
        in_specs=[
            pl.BlockSpec((D, CB), lambda j: (0, j)),
            pl.BlockSpec((D, HIDDEN), lambda j: (0, 0)),
        ],
        out_specs=pl.BlockSpec((SUB, PD), lambda j: (j, 0)),
        out_shape=jax.ShapeDtypeStruct((ROWS, PD), jnp.int32),
        compiler_params=pltpu.CompilerParams(
            fuse_transposed_lhs_in_matmul=True),
    )(tabT, w)


def _sc_gather(gidx2d, tab):
    mesh = plsc.VectorSubcoreMesh(core_axis_name="c", subcore_axis_name="s")

    @functools.partial(
        pl.kernel,
        mesh=mesh,
        out_type=jax.ShapeDtypeStruct((B, PD), jnp.int32),
        scratch_types=[
            pltpu.VMEM((NCH, CHUNK), jnp.int32),
            pltpu.VMEM((2, CHUNK, PD), jnp.int32),
            pltpu.SemaphoreType.DMA,
            pltpu.SemaphoreType.DMA,
            pltpu.SemaphoreType.DMA,
            pltpu.SemaphoreType.DMA,
        ],
    )
    def gather_kernel(idx_hbm, tab_hbm, out_hbm,
                      idx_v, buf_v, gsem0, gsem1, osem0, osem1):
        wid = lax.axis_index("s") * NC + lax.axis_index("c")
        pltpu.sync_copy(idx_hbm.at[pl.ds(wid * NCH, NCH)], idx_v)
        base = wid * BPW
        gsems = (gsem0, gsem1)
        osems = (osem0, osem1)

        def gath(j):
            return pltpu.async_copy(
                tab_hbm.at[idx_v.at[j]], buf_v.at[j % 2], gsems[j % 2])

        def out(j):
            return pltpu.async_copy(
                buf_v.at[j % 2],
                out_hbm.at[pl.ds(base + j * CHUNK, CHUNK)], osems[j % 2])

        gc = [None] * NCH
        oc = [None] * NCH
        gc[0] = gath(0)
        gc[1] = gath(1)
        for j in range(NCH):
            gc[j].wait()
            oc[j] = out(j)
            if j + 2 < NCH:
                oc[j].wait()
                gc[j + 2] = gath(j + 2)
        for j in range(NCH - 2, NCH):
            oc[j].wait()

    return gather_kernel(gidx2d, tab)


def _unpack4(p128_ref):
    vals = []
    for a in range(2):
        h = lax.bitcast_convert_type(
            p128_ref[:, a * HIDDEN:(a + 1) * HIDDEN], jnp.uint32)
        vals.append(lax.bitcast_convert_type(h << 16, jnp.float32))
        vals.append(lax.bitcast_convert_type(
            h & jnp.uint32(0xFFFF0000), jnp.float32))
    return vals


def _mlp_body(u128_ref, d128_ref, ku_ref, kd_ref, img_ref,
              w1i_ref, b1_ref, w2t_ref, b2_ref, out_ref):
    ku = ku_ref[...]
    kd = kd_ref[...]
    acc = jnp.dot(img_ref[...], w1i_ref[...], preferred_element_type=jnp.float32)
    uq = _unpack4(u128_ref)
    dq = _unpack4(d128_ref)
    for k in range(4):
        acc = acc + jnp.where(ku == k, uq[k], 0.0)
        acc = acc + jnp.where(kd == k, dq[k], 0.0)
    h = jnp.maximum(acc + b1_ref[...], 0.0)
    out_ref[...] = jnp.sum(h * w2t_ref[...], axis=1) + b2_ref[0]


def kernel(user_idx, deal_idx, image_vec, user_table, deal_table, W1, b1, W2, b2):
    uidx = user_idx.astype(jnp.int32)
    didx = deal_idx.astype(jnp.int32)
    ugidx2d = ((uidx // CB) * SUB + uidx % SUB).reshape(B // CHUNK, CHUNK)
    dgidx2d = ((didx // CB) * SUB + didx % SUB).reshape(B // CHUNK, CHUNK)

    w1u, w1d, w1i = W1[:D], W1[D:2 * D], W1[2 * D:]
    utw = _premul(user_table.T, w1u)
    u128 = _sc_gather(ugidx2d, utw)
    dtw = _premul(deal_table.T, w1d)
    d128 = _sc_gather(dgidx2d, dtw)

    ku2d = ((uidx % CB) // SUB).reshape(B, 1)
    kd2d = ((didx % CB) // SUB).reshape(B, 1)
    b1r = b1.reshape(1, HIDDEN)
    w2t = W2.reshape(1, HIDDEN)

    score = pl.pallas_call(
        _mlp_body,
        grid=(B // MB,),
        in_specs=[
            pl.BlockSpec((MB, PD), lambda i: (i, 0)),
            pl.BlockSpec((MB, PD), lambda i: (i, 0)),
            pl.BlockSpec((MB, 1), lambda i: (i, 0)),
            pl.BlockSpec((MB, 1), lambda i: (i, 0)),
            pl.BlockSpec((MB, D), lambda i: (i, 0)),
            pl.BlockSpec((D, HIDDEN), lambda i: (0, 0)),
            pl.BlockSpec((1, HIDDEN), lambda i: (0, 0)),
            pl.BlockSpec((1, HIDDEN), lambda i: (0, 0)),
            pl.BlockSpec(memory_space=pltpu.SMEM),
        ],
        out_specs=pl.BlockSpec((MB,), lambda i: (i,)),
        out_shape=jax.ShapeDtypeStruct((B,), jnp.float32),
    )(u128, d128, ku2d, kd2d, image_vec, w1i, b1r, w2t, b2)
    return score
